# baseline (speedup 1.0000x reference)
import functools

import jax
import jax.numpy as jnp
from jax.experimental import pallas as pl
from jax.experimental.pallas import tpu as pltpu

K_TOP = 64
INT_MIN = -(2**31)


def _order_i32(z):
    i = jax.lax.bitcast_convert_type(z, jnp.int32)
    return jnp.where(i < 0, jnp.bitwise_xor(jnp.bitwise_not(i), jnp.int32(INT_MIN)), i)


def _enc_kernel(x_ref, w_ref, b_ref, out_ref):
    cx = x_ref[...] - b_ref[...]
    out_ref[...] = jax.lax.dot_general(
        cx, w_ref[...], (((1,), (1,)), ((), ())),
        preferred_element_type=jnp.float32)


def _topk_dec_kernel(zp_ref, wd_ref, x_ref, b_ref,
                     z_ref, xh_ref, rec_ref, s_ref):
    zp = zp_ref[...]
    r, n = zp.shape
    s = _order_i32(zp)
    s_ref[...] = s

    cm = jnp.max(zp.reshape(r, n // 128, 128), axis=2)
    cs = _order_i32(cm)
    cnt0 = jnp.sum((cs >= 0).astype(jnp.int32), axis=1, keepdims=True)
    t0 = jnp.where(cnt0 >= K_TOP, jnp.zeros_like(cnt0), jnp.int32(INT_MIN))

    def mini_body(it, t):
        c = jnp.bitwise_or(t, jnp.int32(1) << (30 - it))
        cnt = jnp.sum((cs >= c).astype(jnp.int32), axis=1, keepdims=True)
        return jnp.where(cnt >= K_TOP, c, t)

    lo_s = jax.lax.fori_loop(0, 31, mini_body, t0)
    m1_s = _order_i32(jnp.max(cm, axis=1, keepdims=True))
    imin = jnp.int32(INT_MIN)
    lo_b = jnp.bitwise_xor(lo_s, imin)
    hi_b = jnp.bitwise_xor(m1_s, imin) + 1

    ones = jnp.ones((n, 8), jnp.float32)

    def cond(carry):
        it, lo_b, hi_b, ct = carry
        diff = hi_b - lo_b
        wide = jnp.bitwise_and(diff, jnp.int32(-2)) != 0
        act = jnp.logical_and(wide, ct != K_TOP)
        return jnp.logical_and(it < 34, jnp.any(act))

    def body(carry):
        it, lo_b, hi_b, ct = carry
        diff = hi_b - lo_b
        wide = jnp.bitwise_and(diff, jnp.int32(-2)) != 0
        act = jnp.logical_and(wide, ct != K_TOP)
        mid_b = lo_b + jax.lax.shift_right_logical(diff, 1)
        mid_s = jnp.bitwise_xor(mid_b, imin)
        maskf = (s_ref[...] >= mid_s).astype(jnp.float32)
        cnt = jax.lax.dot_general(
            maskf, ones, (((1,), (0,)), ((), ())),
            preferred_element_type=jnp.float32)[:, 0:1].astype(jnp.int32)
        take = jnp.logical_and(cnt >= K_TOP, act)
        drop = jnp.logical_and(cnt < K_TOP, act)
        return (it + 1,
                jnp.where(take, mid_b, lo_b),
                jnp.where(drop, mid_b, hi_b),
                jnp.where(take, cnt, ct))

    big = jnp.full_like(cnt0, n + 1)
    _, lo_b, _, _ = jax.lax.while_loop(
        cond, body, (jnp.int32(0), lo_b, hi_b, big))
    t_s = jnp.bitwise_xor(lo_b, imin)

    z = jnp.where(s_ref[...] >= t_s, zp_ref[...], jnp.zeros_like(zp))
    z_ref[...] = z
    xh = jax.lax.dot_general(
        z.astype(jnp.bfloat16), wd_ref[...], (((1,), (0,)), ((), ())),
        preferred_element_type=jnp.float32) + b_ref[...]
    xh_ref[...] = xh
    rec_ref[...] = x_ref[...] - xh


@functools.partial(jax.jit, static_argnames=("interpret",))
def kernel(inputs, W_enc, W_dec, b_pre, interpret=False):
    n_tok, d_in = inputs.shape
    n_lat = W_enc.shape[0]
    b2 = b_pre.reshape(1, d_in)

    LT1 = 1024
    zp = pl.pallas_call(
        _enc_kernel,
        grid=(n_lat // LT1,),
        in_specs=[
            pl.BlockSpec((n_tok, d_in), lambda j: (0, 0)),
            pl.BlockSpec((LT1, d_in), lambda j: (j, 0)),
            pl.BlockSpec((1, d_in), lambda j: (0, 0)),
        ],
        out_specs=pl.BlockSpec((n_tok, LT1), lambda j: (0, j)),
        out_shape=jax.ShapeDtypeStruct((n_tok, n_lat), jnp.float32),
        interpret=interpret,
    )(inputs, W_enc, b2)

    RT = 32
    wd16 = W_dec.astype(jnp.bfloat16)
    z, x_hat, recon = pl.pallas_call(
        _topk_dec_kernel,
        grid=(n_tok // RT,),
        in_specs=[
            pl.BlockSpec((RT, n_lat), lambda t: (t, 0)),
            pl.BlockSpec((n_lat, d_in), lambda t: (0, 0)),
            pl.BlockSpec((RT, d_in), lambda t: (t, 0)),
            pl.BlockSpec((1, d_in), lambda t: (0, 0)),
        ],
        out_specs=[
            pl.BlockSpec((RT, n_lat), lambda t: (t, 0)),
            pl.BlockSpec((RT, d_in), lambda t: (t, 0)),
            pl.BlockSpec((RT, d_in), lambda t: (t, 0)),
        ],
        out_shape=[
            jax.ShapeDtypeStruct((n_tok, n_lat), jnp.float32),
            jax.ShapeDtypeStruct((n_tok, d_in), jnp.float32),
            jax.ShapeDtypeStruct((n_tok, d_in), jnp.float32),
        ],
        scratch_shapes=[pltpu.VMEM((RT, n_lat), jnp.int32)],
        interpret=interpret,
    )(zp, wd16, inputs, b2)

    return (x_hat, z, zp, recon)


# speedup vs baseline: 1.1662x; 1.1662x over previous
import functools

import jax
import jax.numpy as jnp
from jax.experimental import pallas as pl
from jax.experimental.pallas import tpu as pltpu

K_TOP = 64
INT_MIN = -(2**31)


def _order_i32(z):
    i = jax.lax.bitcast_convert_type(z, jnp.int32)
    return jnp.where(i < 0, jnp.bitwise_xor(jnp.bitwise_not(i), jnp.int32(INT_MIN)), i)


def _enc_kernel(x_ref, w_ref, b_ref, out_ref):
    cx = x_ref[...] - b_ref[...]
    out_ref[...] = jax.lax.dot_general(
        cx, w_ref[...], (((1,), (1,)), ((), ())),
        preferred_element_type=jnp.float32)


def _topk_dec_kernel(zp_ref, wd_ref, x_ref, b_ref,
                     z_ref, xh_ref, rec_ref, s_ref):
    zp = zp_ref[...]
    r, n = zp.shape
    s = _order_i32(zp)
    s_ref[...] = s

    cm = jnp.max(zp.reshape(r, n // 128, 128), axis=2)
    cs = _order_i32(cm)
    cnt0 = jnp.sum((cs >= 0).astype(jnp.int32), axis=1, keepdims=True)
    t0 = jnp.where(cnt0 >= K_TOP, jnp.zeros_like(cnt0), jnp.int32(INT_MIN))

    def mini_body(it, t):
        c = jnp.bitwise_or(t, jnp.int32(1) << (30 - it))
        cnt = jnp.sum((cs >= c).astype(jnp.int32), axis=1, keepdims=True)
        return jnp.where(cnt >= K_TOP, c, t)

    lo_s = jax.lax.fori_loop(0, 31, mini_body, t0)
    m1_s = _order_i32(jnp.max(cm, axis=1, keepdims=True))
    imin = jnp.int32(INT_MIN)
    lo_b = jnp.bitwise_xor(lo_s, imin)
    hi_b = jnp.bitwise_xor(m1_s, imin) + 1

    def cond(carry):
        it, lo_b, hi_b, ct = carry
        diff = hi_b - lo_b
        wide = jnp.bitwise_and(diff, jnp.int32(-2)) != 0
        act = jnp.logical_and(wide, ct != K_TOP)
        return jnp.logical_and(it < 34, jnp.any(act))

    def body(carry):
        it, lo_b, hi_b, ct = carry
        diff = hi_b - lo_b
        wide = jnp.bitwise_and(diff, jnp.int32(-2)) != 0
        act = jnp.logical_and(wide, ct != K_TOP)
        mid_b = lo_b + jax.lax.shift_right_logical(diff, 1)
        mid_s = jnp.bitwise_xor(mid_b, imin)
        cnt = jnp.sum((s_ref[...] >= mid_s).astype(jnp.int32), axis=1,
                      keepdims=True)
        take = jnp.logical_and(cnt >= K_TOP, act)
        drop = jnp.logical_and(cnt < K_TOP, act)
        return (it + 1,
                jnp.where(take, mid_b, lo_b),
                jnp.where(drop, mid_b, hi_b),
                jnp.where(take, cnt, ct))

    big = jnp.full_like(cnt0, n + 1)
    _, lo_b, _, _ = jax.lax.while_loop(
        cond, body, (jnp.int32(0), lo_b, hi_b, big))
    t_s = jnp.bitwise_xor(lo_b, imin)

    z = jnp.where(s_ref[...] >= t_s, zp_ref[...], jnp.zeros_like(zp))
    z_ref[...] = z
    xh = jax.lax.dot_general(
        z.astype(jnp.bfloat16), wd_ref[...], (((1,), (0,)), ((), ())),
        preferred_element_type=jnp.float32) + b_ref[...]
    xh_ref[...] = xh
    rec_ref[...] = x_ref[...] - xh


@functools.partial(jax.jit, static_argnames=("interpret",))
def kernel(inputs, W_enc, W_dec, b_pre, interpret=False):
    n_tok, d_in = inputs.shape
    n_lat = W_enc.shape[0]
    b2 = b_pre.reshape(1, d_in)

    LT1 = 1024
    zp = pl.pallas_call(
        _enc_kernel,
        grid=(n_lat // LT1,),
        in_specs=[
            pl.BlockSpec((n_tok, d_in), lambda j: (0, 0)),
            pl.BlockSpec((LT1, d_in), lambda j: (j, 0)),
            pl.BlockSpec((1, d_in), lambda j: (0, 0)),
        ],
        out_specs=pl.BlockSpec((n_tok, LT1), lambda j: (0, j)),
        out_shape=jax.ShapeDtypeStruct((n_tok, n_lat), jnp.float32),
        interpret=interpret,
    )(inputs, W_enc, b2)

    RT = 32
    wd16 = W_dec.astype(jnp.bfloat16)
    z, x_hat, recon = pl.pallas_call(
        _topk_dec_kernel,
        grid=(n_tok // RT,),
        in_specs=[
            pl.BlockSpec((RT, n_lat), lambda t: (t, 0)),
            pl.BlockSpec((n_lat, d_in), lambda t: (0, 0)),
            pl.BlockSpec((RT, d_in), lambda t: (t, 0)),
            pl.BlockSpec((1, d_in), lambda t: (0, 0)),
        ],
        out_specs=[
            pl.BlockSpec((RT, n_lat), lambda t: (t, 0)),
            pl.BlockSpec((RT, d_in), lambda t: (t, 0)),
            pl.BlockSpec((RT, d_in), lambda t: (t, 0)),
        ],
        out_shape=[
            jax.ShapeDtypeStruct((n_tok, n_lat), jnp.float32),
            jax.ShapeDtypeStruct((n_tok, d_in), jnp.float32),
            jax.ShapeDtypeStruct((n_tok, d_in), jnp.float32),
        ],
        scratch_shapes=[pltpu.VMEM((RT, n_lat), jnp.int32)],
        interpret=interpret,
    )(zp, wd16, inputs, b2)

    return (x_hat, z, zp, recon)


# speedup vs baseline: 1.5223x; 1.3053x over previous
import functools

import jax
import jax.numpy as jnp
from jax.experimental import pallas as pl
from jax.experimental.pallas import tpu as pltpu

K_TOP = 64
INT_MIN = -(2**31)


def _order_i32(z):
    i = jax.lax.bitcast_convert_type(z, jnp.int32)
    return jnp.where(i < 0, jnp.bitwise_xor(jnp.bitwise_not(i), jnp.int32(INT_MIN)), i)


def _enc_kernel(x_ref, w_ref, b_ref, out_ref):
    cx = x_ref[...] - b_ref[...]
    out_ref[...] = jax.lax.dot_general(
        cx, w_ref[...], (((1,), (1,)), ((), ())),
        preferred_element_type=jnp.float32)


def _topk_kernel(zp_ref, z_ref, s_ref):
    zp = zp_ref[...]
    r, n = zp.shape
    s = _order_i32(zp)
    s_ref[...] = s

    cs = jnp.max(s.reshape(r, n // 128, 128), axis=2)
    cnt0 = jnp.sum((cs >= 0).astype(jnp.int32), axis=1, keepdims=True)
    t0 = jnp.where(cnt0 >= K_TOP, jnp.zeros_like(cnt0), jnp.int32(INT_MIN))

    def mini_body(it, t):
        c = jnp.bitwise_or(t, jnp.int32(1) << (30 - it))
        cnt = jnp.sum((cs >= c).astype(jnp.int32), axis=1, keepdims=True)
        return jnp.where(cnt >= K_TOP, c, t)

    lo_s = jax.lax.fori_loop(0, 31, mini_body, t0)
    m1_s = jnp.max(cs, axis=1, keepdims=True)
    imin = jnp.int32(INT_MIN)
    lo_b = jnp.bitwise_xor(lo_s, imin)
    hi_b = jnp.bitwise_xor(m1_s, imin) + 1

    def cond(carry):
        it, lo_b, hi_b, ct = carry
        diff = hi_b - lo_b
        wide = jnp.bitwise_and(diff, jnp.int32(-2)) != 0
        act = jnp.logical_and(wide, ct != K_TOP)
        return jnp.logical_and(it < 34, jnp.any(act))

    def body(carry):
        it, lo_b, hi_b, ct = carry
        diff = hi_b - lo_b
        wide = jnp.bitwise_and(diff, jnp.int32(-2)) != 0
        act = jnp.logical_and(wide, ct != K_TOP)
        mid_b = lo_b + jax.lax.shift_right_logical(diff, 1)
        mid_s = jnp.bitwise_xor(mid_b, imin)
        cnt = jnp.sum((s_ref[...] >= mid_s).astype(jnp.int32), axis=1,
                      keepdims=True)
        take = jnp.logical_and(cnt >= K_TOP, act)
        drop = jnp.logical_and(cnt < K_TOP, act)
        return (it + 1,
                jnp.where(take, mid_b, lo_b),
                jnp.where(drop, mid_b, hi_b),
                jnp.where(take, cnt, ct))

    big = jnp.full_like(cnt0, n + 1)
    _, lo_b, _, _ = jax.lax.while_loop(
        cond, body, (jnp.int32(0), lo_b, hi_b, big))
    t_s = jnp.bitwise_xor(lo_b, imin)

    z_ref[...] = jnp.where(s_ref[...] >= t_s, zp_ref[...],
                           jnp.zeros_like(zp))


def _dec_kernel(z_ref, wd_ref, x_ref, b_ref, xh_ref, rec_ref):
    j = pl.program_id(0)
    nj = pl.num_programs(0)
    contrib = jax.lax.dot_general(
        z_ref[...].astype(jnp.bfloat16), wd_ref[...],
        (((1,), (0,)), ((), ())),
        preferred_element_type=jnp.float32)

    @pl.when(j == 0)
    def _():
        xh_ref[...] = contrib

    @pl.when(j > 0)
    def _():
        xh_ref[...] = xh_ref[...] + contrib

    @pl.when(j == nj - 1)
    def _():
        xh = xh_ref[...] + b_ref[...]
        xh_ref[...] = xh
        rec_ref[...] = x_ref[...] - xh


@functools.partial(jax.jit, static_argnames=("interpret",))
def kernel(inputs, W_enc, W_dec, b_pre, interpret=False):
    n_tok, d_in = inputs.shape
    n_lat = W_enc.shape[0]
    b2 = b_pre.reshape(1, d_in)

    LT1 = 1024
    zp = pl.pallas_call(
        _enc_kernel,
        grid=(n_lat // LT1,),
        in_specs=[
            pl.BlockSpec((n_tok, d_in), lambda j: (0, 0)),
            pl.BlockSpec((LT1, d_in), lambda j: (j, 0)),
            pl.BlockSpec((1, d_in), lambda j: (0, 0)),
        ],
        out_specs=pl.BlockSpec((n_tok, LT1), lambda j: (0, j)),
        out_shape=jax.ShapeDtypeStruct((n_tok, n_lat), jnp.float32),
        interpret=interpret,
    )(inputs, W_enc, b2)

    RT = 128
    z = pl.pallas_call(
        _topk_kernel,
        grid=(n_tok // RT,),
        in_specs=[pl.BlockSpec((RT, n_lat), lambda t: (t, 0))],
        out_specs=pl.BlockSpec((RT, n_lat), lambda t: (t, 0)),
        out_shape=jax.ShapeDtypeStruct((n_tok, n_lat), jnp.float32),
        scratch_shapes=[pltpu.VMEM((RT, n_lat), jnp.int32)],
        interpret=interpret,
    )(zp)

    LT3 = 512
    wd16 = W_dec.astype(jnp.bfloat16)
    x_hat, recon = pl.pallas_call(
        _dec_kernel,
        grid=(n_lat // LT3,),
        in_specs=[
            pl.BlockSpec((n_tok, LT3), lambda j: (0, j)),
            pl.BlockSpec((LT3, d_in), lambda j: (j, 0)),
            pl.BlockSpec((n_tok, d_in), lambda j: (0, 0)),
            pl.BlockSpec((1, d_in), lambda j: (0, 0)),
        ],
        out_specs=[
            pl.BlockSpec((n_tok, d_in), lambda j: (0, 0)),
            pl.BlockSpec((n_tok, d_in), lambda j: (0, 0)),
        ],
        out_shape=[
            jax.ShapeDtypeStruct((n_tok, d_in), jnp.float32),
            jax.ShapeDtypeStruct((n_tok, d_in), jnp.float32),
        ],
        interpret=interpret,
    )(z, wd16, inputs, b2)

    return (x_hat, z, zp, recon)


# speedup vs baseline: 4.6636x; 3.0636x over previous
import functools

import jax
import jax.numpy as jnp
from jax.experimental import pallas as pl
from jax.experimental.pallas import tpu as pltpu

K_TOP = 64
INT_MIN = -(2**31)


def _order_i32(z):
    i = jax.lax.bitcast_convert_type(z, jnp.int32)
    return jnp.where(i < 0, jnp.bitwise_xor(jnp.bitwise_not(i), jnp.int32(INT_MIN)), i)


def _enc_kernel(x_ref, w_ref, b_ref, out_ref):
    cx = x_ref[...] - b_ref[...]
    out_ref[...] = jax.lax.dot_general(
        cx, w_ref[...], (((1,), (1,)), ((), ())),
        preferred_element_type=jnp.float32)


def _topk_kernel(zp_ref, z_ref, s_ref):
    zp = zp_ref[...]
    r, n = zp.shape
    s = _order_i32(zp)
    s_ref[...] = s

    gmax = jnp.max(s.reshape(r, n // 128, 128), axis=1)
    m1_s = jnp.max(gmax, axis=1, keepdims=True)
    lo_s = jnp.min(gmax, axis=1, keepdims=True)
    imin = jnp.int32(INT_MIN)
    lo_b = jnp.bitwise_xor(lo_s, imin)
    hi_b = jnp.bitwise_xor(m1_s, imin) + 1

    def cond(carry):
        it, lo_b, hi_b, ct = carry
        diff = hi_b - lo_b
        wide = jnp.bitwise_and(diff, jnp.int32(-2)) != 0
        act = jnp.logical_and(wide, ct != K_TOP)
        return jnp.logical_and(it < 34, jnp.any(act))

    def body(carry):
        it, lo_b, hi_b, ct = carry
        diff = hi_b - lo_b
        wide = jnp.bitwise_and(diff, jnp.int32(-2)) != 0
        act = jnp.logical_and(wide, ct != K_TOP)
        mid_b = lo_b + jax.lax.shift_right_logical(diff, 1)
        mid_s = jnp.bitwise_xor(mid_b, imin)
        cnt = jnp.sum((s_ref[...] >= mid_s).astype(jnp.int32), axis=1,
                      keepdims=True)
        take = jnp.logical_and(cnt >= K_TOP, act)
        drop = jnp.logical_and(cnt < K_TOP, act)
        return (it + 1,
                jnp.where(take, mid_b, lo_b),
                jnp.where(drop, mid_b, hi_b),
                jnp.where(take, cnt, ct))

    big = jnp.full_like(lo_b, n + 1)
    _, lo_b, _, _ = jax.lax.while_loop(
        cond, body, (jnp.int32(0), lo_b, hi_b, big))
    t_s = jnp.bitwise_xor(lo_b, imin)

    z_ref[...] = jnp.where(s_ref[...] >= t_s, zp_ref[...],
                           jnp.zeros_like(zp))


def _dec_kernel(z_ref, wd_ref, x_ref, b_ref, xh_ref, rec_ref):
    j = pl.program_id(0)
    nj = pl.num_programs(0)
    contrib = jax.lax.dot_general(
        z_ref[...].astype(jnp.bfloat16), wd_ref[...],
        (((1,), (0,)), ((), ())),
        preferred_element_type=jnp.float32)

    @pl.when(j == 0)
    def _():
        xh_ref[...] = contrib

    @pl.when(j > 0)
    def _():
        xh_ref[...] = xh_ref[...] + contrib

    @pl.when(j == nj - 1)
    def _():
        xh = xh_ref[...] + b_ref[...]
        xh_ref[...] = xh
        rec_ref[...] = x_ref[...] - xh


@functools.partial(jax.jit, static_argnames=("interpret",))
def kernel(inputs, W_enc, W_dec, b_pre, interpret=False):
    n_tok, d_in = inputs.shape
    n_lat = W_enc.shape[0]
    b2 = b_pre.reshape(1, d_in)

    LT1 = 1024
    zp = pl.pallas_call(
        _enc_kernel,
        grid=(n_lat // LT1,),
        in_specs=[
            pl.BlockSpec((n_tok, d_in), lambda j: (0, 0)),
            pl.BlockSpec((LT1, d_in), lambda j: (j, 0)),
            pl.BlockSpec((1, d_in), lambda j: (0, 0)),
        ],
        out_specs=pl.BlockSpec((n_tok, LT1), lambda j: (0, j)),
        out_shape=jax.ShapeDtypeStruct((n_tok, n_lat), jnp.float32),
        interpret=interpret,
    )(inputs, W_enc, b2)

    RT = 128
    z = pl.pallas_call(
        _topk_kernel,
        grid=(n_tok // RT,),
        in_specs=[pl.BlockSpec((RT, n_lat), lambda t: (t, 0))],
        out_specs=pl.BlockSpec((RT, n_lat), lambda t: (t, 0)),
        out_shape=jax.ShapeDtypeStruct((n_tok, n_lat), jnp.float32),
        scratch_shapes=[pltpu.VMEM((RT, n_lat), jnp.int32)],
        interpret=interpret,
    )(zp)

    LT3 = 512
    wd16 = W_dec.astype(jnp.bfloat16)
    x_hat, recon = pl.pallas_call(
        _dec_kernel,
        grid=(n_lat // LT3,),
        in_specs=[
            pl.BlockSpec((n_tok, LT3), lambda j: (0, j)),
            pl.BlockSpec((LT3, d_in), lambda j: (j, 0)),
            pl.BlockSpec((n_tok, d_in), lambda j: (0, 0)),
            pl.BlockSpec((1, d_in), lambda j: (0, 0)),
        ],
        out_specs=[
            pl.BlockSpec((n_tok, d_in), lambda j: (0, 0)),
            pl.BlockSpec((n_tok, d_in), lambda j: (0, 0)),
        ],
        out_shape=[
            jax.ShapeDtypeStruct((n_tok, d_in), jnp.float32),
            jax.ShapeDtypeStruct((n_tok, d_in), jnp.float32),
        ],
        interpret=interpret,
    )(z, wd16, inputs, b2)

    return (x_hat, z, zp, recon)


# speedup vs baseline: 4.7787x; 1.0247x over previous
import functools

import jax
import jax.numpy as jnp
from jax.experimental import pallas as pl
from jax.experimental.pallas import tpu as pltpu

K_TOP = 64
INT_MIN = -(2**31)


def _order_i32(z):
    i = jax.lax.bitcast_convert_type(z, jnp.int32)
    return jnp.where(i < 0, jnp.bitwise_xor(jnp.bitwise_not(i), jnp.int32(INT_MIN)), i)


def _enc_kernel(x_ref, w_ref, b_ref, out_ref):
    cx = x_ref[...] - b_ref[...]
    out_ref[...] = jax.lax.dot_general(
        cx, w_ref[...], (((1,), (1,)), ((), ())),
        preferred_element_type=jnp.float32)


def _topk_kernel(zp_ref, z_ref, s_ref):
    zp = zp_ref[...]
    r, n = zp.shape
    s = _order_i32(zp)
    s_ref[...] = s

    gmax = jnp.max(s.reshape(r, n // 128, 128), axis=1)
    m1_s = jnp.max(gmax, axis=1, keepdims=True)
    lo_s = jnp.min(gmax, axis=1, keepdims=True)
    imin = jnp.int32(INT_MIN)
    lo_b = jnp.bitwise_xor(lo_s, imin)
    hi_b = jnp.bitwise_xor(m1_s, imin) + 1

    def cond(carry):
        it, lo_b, hi_b, ct = carry
        diff = hi_b - lo_b
        wide = jnp.bitwise_and(diff, jnp.int32(-2)) != 0
        act = jnp.logical_and(wide, ct != K_TOP)
        return jnp.logical_and(it < 34, jnp.any(act))

    def body(carry):
        it, lo_b, hi_b, ct = carry
        diff = hi_b - lo_b
        wide = jnp.bitwise_and(diff, jnp.int32(-2)) != 0
        act = jnp.logical_and(wide, ct != K_TOP)
        mid_b = lo_b + jax.lax.shift_right_logical(diff, 1)
        mid_s = jnp.bitwise_xor(mid_b, imin)
        cnt = jnp.sum((s_ref[...] >= mid_s).astype(jnp.int32), axis=1,
                      keepdims=True)
        take = jnp.logical_and(cnt >= K_TOP, act)
        drop = jnp.logical_and(cnt < K_TOP, act)
        return (it + 1,
                jnp.where(take, mid_b, lo_b),
                jnp.where(drop, mid_b, hi_b),
                jnp.where(take, cnt, ct))

    big = jnp.full_like(lo_b, n + 1)
    _, lo_b, _, _ = jax.lax.while_loop(
        cond, body, (jnp.int32(0), lo_b, hi_b, big))
    t_s = jnp.bitwise_xor(lo_b, imin)

    z_ref[...] = jnp.where(s_ref[...] >= t_s, zp_ref[...],
                           jnp.zeros_like(zp))


def _dec_kernel(z_ref, wd_ref, x_ref, b_ref, xh_ref, rec_ref):
    j = pl.program_id(0)
    nj = pl.num_programs(0)
    contrib = jax.lax.dot_general(
        z_ref[...].astype(jnp.bfloat16), wd_ref[...],
        (((1,), (0,)), ((), ())),
        preferred_element_type=jnp.float32)

    @pl.when(j == 0)
    def _():
        xh_ref[...] = contrib

    @pl.when(j > 0)
    def _():
        xh_ref[...] = xh_ref[...] + contrib

    @pl.when(j == nj - 1)
    def _():
        xh = xh_ref[...] + b_ref[...]
        xh_ref[...] = xh
        rec_ref[...] = x_ref[...] - xh


@functools.partial(jax.jit, static_argnames=("interpret",))
def kernel(inputs, W_enc, W_dec, b_pre, interpret=False):
    n_tok, d_in = inputs.shape
    n_lat = W_enc.shape[0]
    b2 = b_pre.reshape(1, d_in)

    LT1 = 1024
    zp = pl.pallas_call(
        _enc_kernel,
        grid=(n_lat // LT1,),
        in_specs=[
            pl.BlockSpec((n_tok, d_in), lambda j: (0, 0)),
            pl.BlockSpec((LT1, d_in), lambda j: (j, 0)),
            pl.BlockSpec((1, d_in), lambda j: (0, 0)),
        ],
        out_specs=pl.BlockSpec((n_tok, LT1), lambda j: (0, j)),
        out_shape=jax.ShapeDtypeStruct((n_tok, n_lat), jnp.float32),
        interpret=interpret,
    )(inputs, W_enc, b2)

    RT = 128
    z = pl.pallas_call(
        _topk_kernel,
        grid=(n_tok // RT,),
        in_specs=[pl.BlockSpec((RT, n_lat), lambda t: (t, 0))],
        out_specs=pl.BlockSpec((RT, n_lat), lambda t: (t, 0)),
        out_shape=jax.ShapeDtypeStruct((n_tok, n_lat), jnp.float32),
        scratch_shapes=[pltpu.VMEM((RT, n_lat), jnp.int32)],
        interpret=interpret,
    )(zp)

    LT3 = 1024
    wd16 = W_dec.astype(jnp.bfloat16)
    x_hat, recon = pl.pallas_call(
        _dec_kernel,
        grid=(n_lat // LT3,),
        in_specs=[
            pl.BlockSpec((n_tok, LT3), lambda j: (0, j)),
            pl.BlockSpec((LT3, d_in), lambda j: (j, 0)),
            pl.BlockSpec((n_tok, d_in), lambda j: (0, 0)),
            pl.BlockSpec((1, d_in), lambda j: (0, 0)),
        ],
        out_specs=[
            pl.BlockSpec((n_tok, d_in), lambda j: (0, 0)),
            pl.BlockSpec((n_tok, d_in), lambda j: (0, 0)),
        ],
        out_shape=[
            jax.ShapeDtypeStruct((n_tok, d_in), jnp.float32),
            jax.ShapeDtypeStruct((n_tok, d_in), jnp.float32),
        ],
        interpret=interpret,
    )(z, wd16, inputs, b2)

    return (x_hat, z, zp, recon)


# speedup vs baseline: 4.7795x; 1.0002x over previous
import jax
import jax.numpy as jnp
from jax.experimental import pallas as pl
from jax.experimental.pallas import tpu as pltpu

K_TOP = 64
INT_MIN = -(2**31)


def _order_i32(z):
    i = jax.lax.bitcast_convert_type(z, jnp.int32)
    return jnp.where(i < 0, jnp.bitwise_xor(jnp.bitwise_not(i), jnp.int32(INT_MIN)), i)


def _enc_kernel(x_ref, w_ref, b_ref, out_ref):
    cx = x_ref[...] - b_ref[...]
    out_ref[...] = jax.lax.dot_general(
        cx, w_ref[...], (((1,), (1,)), ((), ())),
        preferred_element_type=jnp.float32)


def _topk_kernel(zp_ref, z_ref, s_ref):
    zp = zp_ref[...]
    r, n = zp.shape
    s = _order_i32(zp)
    s_ref[...] = s

    gmax = jnp.max(s.reshape(r, n // 128, 128), axis=1)
    m1_s = jnp.max(gmax, axis=1, keepdims=True)
    lo_s = jnp.min(gmax, axis=1, keepdims=True)
    imin = jnp.int32(INT_MIN)
    lo_b = jnp.bitwise_xor(lo_s, imin)
    hi_b = jnp.bitwise_xor(m1_s, imin) + 1

    def cond(carry):
        it, lo_b, hi_b, ct = carry
        diff = hi_b - lo_b
        wide = jnp.bitwise_and(diff, jnp.int32(-2)) != 0
        act = jnp.logical_and(wide, ct != K_TOP)
        return jnp.logical_and(it < 34, jnp.any(act))

    def body(carry):
        it, lo_b, hi_b, ct = carry
        diff = hi_b - lo_b
        wide = jnp.bitwise_and(diff, jnp.int32(-2)) != 0
        act = jnp.logical_and(wide, ct != K_TOP)
        mid_b = lo_b + jax.lax.shift_right_logical(diff, 1)
        mid_s = jnp.bitwise_xor(mid_b, imin)
        cnt = jnp.sum((s_ref[...] >= mid_s).astype(jnp.int32), axis=1,
                      keepdims=True)
        take = jnp.logical_and(cnt >= K_TOP, act)
        drop = jnp.logical_and(cnt < K_TOP, act)
        return (it + 1,
                jnp.where(take, mid_b, lo_b),
                jnp.where(drop, mid_b, hi_b),
                jnp.where(take, cnt, ct))

    big = jnp.full_like(lo_b, n + 1)
    _, lo_b, _, _ = jax.lax.while_loop(
        cond, body, (jnp.int32(0), lo_b, hi_b, big))
    t_s = jnp.bitwise_xor(lo_b, imin)

    z_ref[...] = jnp.where(s_ref[...] >= t_s, zp_ref[...],
                           jnp.zeros_like(zp))


def _dec_kernel(z_ref, wd_ref, x_ref, b_ref, xh_ref, rec_ref):
    j = pl.program_id(0)
    nj = pl.num_programs(0)
    contrib = jax.lax.dot_general(
        z_ref[...].astype(jnp.bfloat16), wd_ref[...],
        (((1,), (0,)), ((), ())),
        preferred_element_type=jnp.float32)

    @pl.when(j == 0)
    def _():
        xh_ref[...] = contrib

    @pl.when(j > 0)
    def _():
        xh_ref[...] = xh_ref[...] + contrib

    @pl.when(j == nj - 1)
    def _():
        xh = xh_ref[...] + b_ref[...]
        xh_ref[...] = xh
        rec_ref[...] = x_ref[...] - xh


@jax.jit
def kernel(inputs, W_enc, W_dec, b_pre):
    n_tok, d_in = inputs.shape
    n_lat = W_enc.shape[0]
    b2 = b_pre.reshape(1, d_in)

    LT1 = 1024
    zp = pl.pallas_call(
        _enc_kernel,
        grid=(n_lat // LT1,),
        in_specs=[
            pl.BlockSpec((n_tok, d_in), lambda j: (0, 0)),
            pl.BlockSpec((LT1, d_in), lambda j: (j, 0)),
            pl.BlockSpec((1, d_in), lambda j: (0, 0)),
        ],
        out_specs=pl.BlockSpec((n_tok, LT1), lambda j: (0, j)),
        out_shape=jax.ShapeDtypeStruct((n_tok, n_lat), jnp.float32),
    )(inputs, W_enc, b2)

    RT = 128
    z = pl.pallas_call(
        _topk_kernel,
        grid=(n_tok // RT,),
        in_specs=[pl.BlockSpec((RT, n_lat), lambda t: (t, 0))],
        out_specs=pl.BlockSpec((RT, n_lat), lambda t: (t, 0)),
        out_shape=jax.ShapeDtypeStruct((n_tok, n_lat), jnp.float32),
        scratch_shapes=[pltpu.VMEM((RT, n_lat), jnp.int32)],
    )(zp)

    LT3 = 1024
    wd16 = W_dec.astype(jnp.bfloat16)
    x_hat, recon = pl.pallas_call(
        _dec_kernel,
        grid=(n_lat // LT3,),
        in_specs=[
            pl.BlockSpec((n_tok, LT3), lambda j: (0, j)),
            pl.BlockSpec((LT3, d_in), lambda j: (j, 0)),
            pl.BlockSpec((n_tok, d_in), lambda j: (0, 0)),
            pl.BlockSpec((1, d_in), lambda j: (0, 0)),
        ],
        out_specs=[
            pl.BlockSpec((n_tok, d_in), lambda j: (0, 0)),
            pl.BlockSpec((n_tok, d_in), lambda j: (0, 0)),
        ],
        out_shape=[
            jax.ShapeDtypeStruct((n_tok, d_in), jnp.float32),
            jax.ShapeDtypeStruct((n_tok, d_in), jnp.float32),
        ],
    )(z, wd16, inputs, b2)

    return (x_hat, z, zp, recon)
